# baseline (speedup 1.0000x reference)
import jax
import jax.numpy as jnp
from jax import lax
from jax.experimental import pallas as pl
from jax.experimental.pallas import tpu as pltpu
from jax.experimental.pallas import tpu_sc as plsc

EMBED_DIM = 64
BATCH = 16384
N_FIELDS = 26
NUM_FEAT = 1000000

NUM_CORES = 2
NUM_SUBCORES = 16
NUM_WORKERS = NUM_CORES * NUM_SUBCORES

FB = 256
FULL_FEAT = 999936
N_BANDS = FULL_FEAT // FB
A_ROUNDS = (N_BANDS + 2 * NUM_WORKERS - 1) // (2 * NUM_WORKERS)

CB = 128
N_BLOCKS = N_FIELDS * (BATCH // CB)
BLOCKS_PER_WORKER = N_BLOCKS // NUM_WORKERS
B_ROUNDS = BLOCKS_PER_WORKER // 2
TCOLS = BATCH // CB


def _detile_body(wt_hbm, tail_hbm, wp_hbm,
                 in0, in1, po0, po1, tl_v, s0, s1):
    wid = lax.axis_index("s") * NUM_CORES + lax.axis_index("c")
    iota = lax.iota(jnp.int32, 16)

    @pl.when(wid == 0)
    def _():
        pltpu.sync_copy(tail_hbm, tl_v)
        pltpu.sync_copy(tl_v, wp_hbm.at[pl.ds(FULL_FEAT // 2, 32), :])

    def transpose_band(in_v, po_v):
        def grp(g, c2):
            pp_vec = 16 * g + iota
            for par in range(2):
                src_col = 2 * pp_vec + par
                for j in range(EMBED_DIM):
                    vals = plsc.load_gather(
                        in_v, [jnp.full((16,), j, jnp.int32), src_col])
                    plsc.store_scatter(
                        po_v, [pp_vec, jnp.full((16,), par * EMBED_DIM + j,
                                                jnp.int32)], vals)
            return c2

        lax.fori_loop(0, FB // 2 // 16, grp, 0)

    def round_step(t, carry):
        b0 = wid + NUM_WORKERS * (2 * t)
        b1 = wid + NUM_WORKERS * (2 * t + 1)

        @pl.when(b0 < N_BANDS)
        def _():
            c0 = pltpu.async_copy(
                wt_hbm.at[:, pl.ds(FB * b0, FB)], in0, s0)

            @pl.when(b1 < N_BANDS)
            def _():
                c1 = pltpu.async_copy(
                    wt_hbm.at[:, pl.ds(FB * b1, FB)], in1, s1)
                c0.wait()
                transpose_band(in0, po0)
                pltpu.sync_copy(po0, wp_hbm.at[pl.ds((FB // 2) * b0, FB // 2), :])
                c1.wait()
                transpose_band(in1, po1)
                pltpu.sync_copy(po1, wp_hbm.at[pl.ds((FB // 2) * b1, FB // 2), :])

            @pl.when(b1 >= N_BANDS)
            def _():
                c0.wait()
                transpose_band(in0, po0)
                pltpu.sync_copy(po0, wp_hbm.at[pl.ds((FB // 2) * b0, FB // 2), :])

        return carry

    lax.fori_loop(0, A_ROUNDS, round_step, 0)


def _gather_body(wp_hbm, idx_hbm, out_hbm,
                 idx_all, pidx0, pidx1, col0, col1,
                 rows0, rows1, out0, out1, g0, g1):
    wid = lax.axis_index("s") * NUM_CORES + lax.axis_index("c")
    iota = lax.iota(jnp.int32, 16)

    pltpu.sync_copy(idx_hbm.at[pl.ds(wid * BLOCKS_PER_WORKER,
                                     BLOCKS_PER_WORKER), :], idx_all)

    def prep(tloc, pidx_v, col_v):
        def p(g, c2):
            v = idx_all[tloc, pl.ds(16 * g, 16)]
            pidx_v[pl.ds(16 * g, 16)] = lax.shift_right_logical(v, 1)
            col_v[pl.ds(16 * g, 16)] = lax.shift_left(lax.bitwise_and(v, 1), 6)
            return c2

        lax.fori_loop(0, CB // 16, p, 0)

    def transpose_block(rows_v, col_v, out_v):
        def grp(g, c2):
            c_vec = 16 * g + iota
            base_col = col_v[pl.ds(16 * g, 16)]
            for j in range(EMBED_DIM):
                vals = plsc.load_gather(rows_v, [c_vec, base_col + j])
                out_v[j // 8, j % 8, pl.ds(16 * g, 16)] = vals
            return c2

        lax.fori_loop(0, CB // 16, grp, 0)

    def store(out_v, k):
        f = lax.shift_right_logical(k, 7)
        tc = lax.bitwise_and(k, TCOLS - 1)
        pltpu.sync_copy(out_v, out_hbm.at[f, :, tc])

    def round_step(t, carry):
        t0 = 2 * t
        t1 = 2 * t + 1
        k0 = wid * BLOCKS_PER_WORKER + t0
        k1 = wid * BLOCKS_PER_WORKER + t1

        prep(t0, pidx0, col0)
        c0 = pltpu.async_copy(wp_hbm.at[pidx0], rows0, g0)
        prep(t1, pidx1, col1)
        c1 = pltpu.async_copy(wp_hbm.at[pidx1], rows1, g1)
        c0.wait()
        transpose_block(rows0, col0, out0)
        store(out0, k0)
        c1.wait()
        transpose_block(rows1, col1, out1)
        store(out1, k1)
        return carry

    lax.fori_loop(0, B_ROUNDS, round_step, 0)


def kernel(input_indices, W):
    wt = W.T
    tail_pairs = W[FULL_FEAT:].reshape(32, 2 * EMBED_DIM)
    idx2d = input_indices.T.astype(jnp.int32).reshape(N_BLOCKS, CB)
    mesh = plsc.VectorSubcoreMesh(core_axis_name="c", subcore_axis_name="s")

    w_pairs = pl.kernel(
        _detile_body,
        out_type=jax.ShapeDtypeStruct((NUM_FEAT // 2, 2 * EMBED_DIM),
                                      jnp.float32),
        mesh=mesh,
        scratch_types=[
            pltpu.VMEM((EMBED_DIM, FB), jnp.float32),
            pltpu.VMEM((EMBED_DIM, FB), jnp.float32),
            pltpu.VMEM((FB // 2, 2 * EMBED_DIM), jnp.float32),
            pltpu.VMEM((FB // 2, 2 * EMBED_DIM), jnp.float32),
            pltpu.VMEM((32, 2 * EMBED_DIM), jnp.float32),
            pltpu.SemaphoreType.DMA,
            pltpu.SemaphoreType.DMA,
        ],
        compiler_params=pltpu.CompilerParams(
            needs_layout_passes=False, use_tc_tiling_on_sc=True),
    )(wt, tail_pairs)

    out5d = pl.kernel(
        _gather_body,
        out_type=jax.ShapeDtypeStruct(
            (N_FIELDS, EMBED_DIM // 8, TCOLS, 8, CB), jnp.float32),
        mesh=mesh,
        scratch_types=[
            pltpu.VMEM((BLOCKS_PER_WORKER, CB), jnp.int32),
            pltpu.VMEM((CB,), jnp.int32),
            pltpu.VMEM((CB,), jnp.int32),
            pltpu.VMEM((CB,), jnp.int32),
            pltpu.VMEM((CB,), jnp.int32),
            pltpu.VMEM((CB, 2 * EMBED_DIM), jnp.float32),
            pltpu.VMEM((CB, 2 * EMBED_DIM), jnp.float32),
            pltpu.VMEM((EMBED_DIM // 8, 8, CB), jnp.float32),
            pltpu.VMEM((EMBED_DIM // 8, 8, CB), jnp.float32),
            pltpu.SemaphoreType.DMA,
            pltpu.SemaphoreType.DMA,
        ],
        compiler_params=pltpu.CompilerParams(needs_layout_passes=False),
    )(w_pairs, idx2d)

    return out5d.transpose((2, 4, 0, 1, 3)).reshape(BATCH, N_FIELDS, EMBED_DIM)


# speedup vs baseline: 1.3783x; 1.3783x over previous
import jax
import jax.numpy as jnp
from jax import lax
from jax.experimental import pallas as pl
from jax.experimental.pallas import tpu as pltpu
from jax.experimental.pallas import tpu_sc as plsc

EMBED_DIM = 64
BATCH = 16384
N_FIELDS = 26
NUM_FEAT = 1000000

NUM_CORES = 2
NUM_SUBCORES = 16
NUM_WORKERS = NUM_CORES * NUM_SUBCORES

FB = 512
N_BANDS = (NUM_FEAT + FB - 1) // FB

CB = 128
N_BLOCKS = N_FIELDS * (BATCH // CB)
BLOCKS_PER_WORKER = N_BLOCKS // NUM_WORKERS
B_ROUNDS = BLOCKS_PER_WORKER // 2
TCOLS = BATCH // CB


def _overlap_tc(wt_ref, wt_next_ref, out_ref):
    y = wt_ref[...].T
    ynext = wt_next_ref[...].T
    shifted = jnp.concatenate([y[1:], ynext[:1]], axis=0)
    out_ref[...] = jnp.concatenate([y, shifted], axis=1)


def _gather_body(wo_hbm, idx_hbm, out_hbm,
                 idx_all, rows0, rows1, g0, g1, s0, s1):
    wid = lax.axis_index("s") * NUM_CORES + lax.axis_index("c")

    pltpu.sync_copy(idx_hbm.at[pl.ds(wid * BLOCKS_PER_WORKER,
                                     BLOCKS_PER_WORKER), :], idx_all)

    def round_step(t, carry):
        t0 = 2 * t
        t1 = 2 * t + 1
        k0 = wid * BLOCKS_PER_WORKER + t0
        k1 = wid * BLOCKS_PER_WORKER + t1
        f0 = k0 // TCOLS
        tc0 = k0 % TCOLS
        f1 = k1 // TCOLS
        tc1 = k1 % TCOLS

        c0 = pltpu.async_copy(wo_hbm.at[idx_all.at[t0]], rows0, g0)
        c1 = pltpu.async_copy(wo_hbm.at[idx_all.at[t1]], rows1, g1)
        c0.wait()
        w0 = pltpu.async_copy(rows0, out_hbm.at[pl.ds(tc0 * CB, CB), f0, :], s0)
        c1.wait()
        w1 = pltpu.async_copy(rows1, out_hbm.at[pl.ds(tc1 * CB, CB), f1, :], s1)
        w0.wait()
        w1.wait()
        return carry

    lax.fori_loop(0, B_ROUNDS, round_step, 0)


def kernel(input_indices, W):
    wt = W.T
    idx2d = input_indices.T.astype(jnp.int32).reshape(N_BLOCKS, CB)
    mesh = plsc.VectorSubcoreMesh(core_axis_name="c", subcore_axis_name="s")

    w_over = pl.pallas_call(
        _overlap_tc,
        grid=(N_BANDS,),
        in_specs=[
            pl.BlockSpec((EMBED_DIM, FB), lambda i: (0, i)),
            pl.BlockSpec((EMBED_DIM, FB),
                         lambda i: (0, jnp.minimum(i + 1, N_BANDS - 1))),
        ],
        out_specs=pl.BlockSpec((FB, 2 * EMBED_DIM), lambda i: (i, 0)),
        out_shape=jax.ShapeDtypeStruct((NUM_FEAT, 2 * EMBED_DIM),
                                       jnp.float32),
        compiler_params=pltpu.CompilerParams(
            dimension_semantics=("arbitrary",)),
    )(wt, wt)

    out4 = pl.kernel(
        _gather_body,
        out_type=jax.ShapeDtypeStruct((BATCH, 32, 2 * EMBED_DIM),
                                      jnp.float32),
        mesh=mesh,
        scratch_types=[
            pltpu.VMEM((BLOCKS_PER_WORKER, CB), jnp.int32),
            pltpu.VMEM((CB, 2 * EMBED_DIM), jnp.float32),
            pltpu.VMEM((CB, 2 * EMBED_DIM), jnp.float32),
            pltpu.SemaphoreType.DMA,
            pltpu.SemaphoreType.DMA,
            pltpu.SemaphoreType.DMA,
            pltpu.SemaphoreType.DMA,
        ],
    )(w_over, idx2d)

    return out4[:, :N_FIELDS, :EMBED_DIM]


# speedup vs baseline: 1.4864x; 1.0784x over previous
import jax
import jax.numpy as jnp
from jax import lax
from jax.experimental import pallas as pl
from jax.experimental.pallas import tpu as pltpu
from jax.experimental.pallas import tpu_sc as plsc

EMBED_DIM = 64
BATCH = 16384
N_FIELDS = 26
NUM_FEAT = 1000000

NUM_CORES = 2
NUM_SUBCORES = 16
NUM_WORKERS = NUM_CORES * NUM_SUBCORES

FB = 512
N_BANDS = (NUM_FEAT + FB - 1) // FB

CB = 128
N_BLOCKS = N_FIELDS * (BATCH // CB)
BLOCKS_PER_WORKER = N_BLOCKS // NUM_WORKERS
B_ROUNDS = BLOCKS_PER_WORKER // 2
TCOLS = BATCH // CB


def _overlap_tc(wt_ref, out_ref):
    out_ref[:, :EMBED_DIM] = wt_ref[...].T


def _gather_body(wo_hbm, idx_hbm, out_hbm,
                 idx_all, rows0, rows1, g0, g1, s0, s1):
    wid = lax.axis_index("s") * NUM_CORES + lax.axis_index("c")

    pltpu.sync_copy(idx_hbm.at[pl.ds(wid * BLOCKS_PER_WORKER,
                                     BLOCKS_PER_WORKER), :], idx_all)

    def round_step(t, carry):
        t0 = 2 * t
        t1 = 2 * t + 1
        k0 = wid * BLOCKS_PER_WORKER + t0
        k1 = wid * BLOCKS_PER_WORKER + t1
        f0 = k0 // TCOLS
        tc0 = k0 % TCOLS
        f1 = k1 // TCOLS
        tc1 = k1 % TCOLS

        c0 = pltpu.async_copy(wo_hbm.at[idx_all.at[t0]], rows0, g0)
        c1 = pltpu.async_copy(wo_hbm.at[idx_all.at[t1]], rows1, g1)
        c0.wait()
        w0 = pltpu.async_copy(rows0, out_hbm.at[pl.ds(tc0 * CB, CB), f0, :], s0)
        c1.wait()
        w1 = pltpu.async_copy(rows1, out_hbm.at[pl.ds(tc1 * CB, CB), f1, :], s1)
        w0.wait()
        w1.wait()
        return carry

    lax.fori_loop(0, B_ROUNDS, round_step, 0)


def kernel(input_indices, W):
    wt = W.T
    idx2d = input_indices.T.astype(jnp.int32).reshape(N_BLOCKS, CB)
    mesh = plsc.VectorSubcoreMesh(core_axis_name="c", subcore_axis_name="s")

    w_over = pl.pallas_call(
        _overlap_tc,
        grid=(N_BANDS,),
        in_specs=[
            pl.BlockSpec((EMBED_DIM, FB), lambda i: (0, i)),
        ],
        out_specs=pl.BlockSpec((FB, 2 * EMBED_DIM), lambda i: (i, 0)),
        out_shape=jax.ShapeDtypeStruct((NUM_FEAT, 2 * EMBED_DIM),
                                       jnp.float32),
        compiler_params=pltpu.CompilerParams(
            dimension_semantics=("arbitrary",)),
    )(wt)

    out4 = pl.kernel(
        _gather_body,
        out_type=jax.ShapeDtypeStruct((BATCH, 32, 2 * EMBED_DIM),
                                      jnp.float32),
        mesh=mesh,
        scratch_types=[
            pltpu.VMEM((BLOCKS_PER_WORKER, CB), jnp.int32),
            pltpu.VMEM((CB, 2 * EMBED_DIM), jnp.float32),
            pltpu.VMEM((CB, 2 * EMBED_DIM), jnp.float32),
            pltpu.SemaphoreType.DMA,
            pltpu.SemaphoreType.DMA,
            pltpu.SemaphoreType.DMA,
            pltpu.SemaphoreType.DMA,
        ],
    )(w_over, idx2d)

    return out4[:, :N_FIELDS, :EMBED_DIM]
